# Initial kernel scaffold; baseline (speedup 1.0000x reference)
import jax
import jax.numpy as jnp
from jax.experimental import pallas as pl


def kernel(x, pos, edge_index, edge_weights, lift_W, lift_b, ker_W1, ker_b1, ker_W2, ker_b2, proj_W, proj_b):
    raise NotImplementedError("write your pallas kernel here")


# speedup vs baseline: 2.3886x; 2.3886x over previous
import functools

import jax
import jax.numpy as jnp
from jax import lax
from jax.experimental import pallas as pl
from jax.experimental.pallas import tpu as pltpu
from jax.experimental.pallas import tpu_sc as plsc

NC = 2
NS = 16
LANES = 16
CH = 128


def _sc_mesh():
    return plsc.VectorSubcoreMesh(core_axis_name="c", subcore_axis_name="s",
                                  num_cores=NC, num_subcores=NS)


def _worker_id():
    c = lax.axis_index("c")
    s = lax.axis_index("s")
    return s * NC + c, c, s


def _sc_prep_body(n_pad, n_chunks, pos_w, pos_hbm, src_hbm, dst_hbm,
                  posd_hbm, poss_hbm, degp_hbm,
                  src_v, dst_v, srows_v, drows_v, ones_v, zeros_v,
                  deg_sh):
    wid, c, s = _worker_id()
    nw = NC * NS
    base_chunks = n_chunks // nw
    rem = n_chunks - base_chunks * nw
    my_chunks = base_chunks + jnp.where(wid < rem, 1, 0)

    rows_per_tile = n_pad // NS

    def fill(i, _):
        ones_v[i, pl.ds(0, LANES)] = jnp.ones((LANES,), jnp.float32)
        zeros_v[i, pl.ds(0, LANES)] = jnp.zeros((LANES,), jnp.float32)
        return 0
    lax.fori_loop(0, CH, fill, 0)
    for b in range(rows_per_tile // CH):
        pltpu.sync_copy(zeros_v, deg_sh.at[pl.ds(s * rows_per_tile + b * CH, CH)])
    plsc.subcore_barrier()

    def chunk_body(t, _):
        cid = wid + nw * t
        ebase = cid * CH
        pltpu.sync_copy(src_hbm.at[pl.ds(ebase, CH)], src_v)
        pltpu.sync_copy(dst_hbm.at[pl.ds(ebase, CH)], dst_v)
        pltpu.sync_copy(pos_hbm.at[src_v], srows_v)
        pltpu.sync_copy(pos_hbm.at[dst_v], drows_v)
        pltpu.sync_copy(srows_v, poss_hbm.at[pl.ds(ebase, CH)])
        pltpu.sync_copy(drows_v, posd_hbm.at[pl.ds(ebase, CH)])
        pltpu.sync_copy(ones_v, deg_sh.at[dst_v], add=True)
        return 0

    lax.fori_loop(0, my_chunks, chunk_body, 0)
    plsc.subcore_barrier()
    pltpu.sync_copy(deg_sh.at[pl.ds(s * rows_per_tile, rows_per_tile)],
                    degp_hbm.at[c, pl.ds(s * rows_per_tile, rows_per_tile)])


def _sc_prep(pos, src, dst, n_pad):
    e = src.shape[0]
    pos_w = pos.shape[1]
    n_chunks = e // CH
    kern = functools.partial(
        pl.kernel,
        out_type=(jax.ShapeDtypeStruct((e, pos_w), jnp.float32),
                  jax.ShapeDtypeStruct((e, pos_w), jnp.float32),
                  jax.ShapeDtypeStruct((NC, n_pad, LANES), jnp.float32)),
        mesh=_sc_mesh(),
        compiler_params=pltpu.CompilerParams(use_tc_tiling_on_sc=False),
        scratch_types=[
            pltpu.VMEM((CH,), jnp.int32),
            pltpu.VMEM((CH,), jnp.int32),
            pltpu.VMEM((CH, pos_w), jnp.float32),
            pltpu.VMEM((CH, pos_w), jnp.float32),
            pltpu.VMEM((CH, LANES), jnp.float32),
            pltpu.VMEM((CH, LANES), jnp.float32),
            pltpu.VMEM_SHARED((n_pad, LANES), jnp.float32),
        ],
    )(functools.partial(_sc_prep_body, n_pad, n_chunks, pos_w))
    return kern(pos, src, dst)


def _sc_layer_body(n_pad, h_dim, n_chunks, h_hbm, kw_hbm, src_hbm, dst_hbm,
                   aggp_hbm,
                   src_v, dst_v, hrows_v, kw_v, zero_v, acc_sh):
    wid, c, s = _worker_id()
    nw = NC * NS
    base_chunks = n_chunks // nw
    rem = n_chunks - base_chunks * nw
    my_chunks = base_chunks + jnp.where(wid < rem, 1, 0)
    rows_per_tile = n_pad // NS

    zero16 = jnp.zeros((LANES,), jnp.float32)

    def zfill(i, _):
        for j in range(h_dim // LANES):
            zero_v[i, pl.ds(j * LANES, LANES)] = zero16
        return 0
    lax.fori_loop(0, CH, zfill, 0)
    for b in range(rows_per_tile // CH):
        pltpu.sync_copy(zero_v, acc_sh.at[pl.ds(s * rows_per_tile + b * CH, CH)])
    plsc.subcore_barrier()

    def chunk_body(t, _):
        cid = wid + nw * t
        ebase = cid * CH
        pltpu.sync_copy(src_hbm.at[pl.ds(ebase, CH)], src_v)
        pltpu.sync_copy(dst_hbm.at[pl.ds(ebase, CH)], dst_v)
        pltpu.sync_copy(h_hbm.at[src_v], hrows_v)
        pltpu.sync_copy(kw_hbm.at[pl.ds(ebase, CH)], kw_v)

        def row(i, _):
            for j in range(h_dim // LANES):
                sl = pl.ds(j * LANES, LANES)
                hrows_v[i, sl] = hrows_v[i, sl] * kw_v[i, sl]
            return 0
        lax.fori_loop(0, CH, row, 0)
        pltpu.sync_copy(hrows_v, acc_sh.at[dst_v], add=True)
        return 0

    lax.fori_loop(0, my_chunks, chunk_body, 0)
    plsc.subcore_barrier()
    pltpu.sync_copy(acc_sh.at[pl.ds(s * rows_per_tile, rows_per_tile)],
                    aggp_hbm.at[c, pl.ds(s * rows_per_tile, rows_per_tile)])


def _sc_layer(h, kw, src, dst, n_pad):
    e = src.shape[0]
    h_dim = h.shape[1]
    n_chunks = e // CH
    kern = functools.partial(
        pl.kernel,
        out_type=jax.ShapeDtypeStruct((NC, n_pad, h_dim), jnp.float32),
        mesh=_sc_mesh(),
        compiler_params=pltpu.CompilerParams(use_tc_tiling_on_sc=False),
        scratch_types=[
            pltpu.VMEM((CH,), jnp.int32),
            pltpu.VMEM((CH,), jnp.int32),
            pltpu.VMEM((CH, h_dim), jnp.float32),
            pltpu.VMEM((CH, h_dim), jnp.float32),
            pltpu.VMEM((CH, h_dim), jnp.float32),
            pltpu.VMEM_SHARED((n_pad, h_dim), jnp.float32),
        ],
    )(functools.partial(_sc_layer_body, n_pad, h_dim, n_chunks))
    return kern(h, kw, src, dst)


def _tc_lift(x2d, pos, wx, wp, b):
    n = x2d.shape[0]
    h_dim = wx.shape[1]
    bn = 2000 if n % 2000 == 0 else n

    def body(x_ref, p_ref, wx_ref, wp_ref, b_ref, o_ref):
        o_ref[...] = (jnp.dot(x_ref[...], wx_ref[...],
                              preferred_element_type=jnp.float32)
                      + jnp.dot(p_ref[...], wp_ref[...],
                                preferred_element_type=jnp.float32)
                      + b_ref[...])

    return pl.pallas_call(
        body,
        grid=(n // bn,),
        in_specs=[
            pl.BlockSpec((bn, x2d.shape[1]), lambda i: (i, 0)),
            pl.BlockSpec((bn, pos.shape[1]), lambda i: (i, 0)),
            pl.BlockSpec(wx.shape, lambda i: (0, 0)),
            pl.BlockSpec(wp.shape, lambda i: (0, 0)),
            pl.BlockSpec((1, h_dim), lambda i: (0, 0)),
        ],
        out_specs=pl.BlockSpec((bn, h_dim), lambda i: (i, 0)),
        out_shape=jax.ShapeDtypeStruct((n, h_dim), jnp.float32),
    )(x2d, pos, wx, wp, b.reshape(1, h_dim))


def _tc_kmlp(posd, poss, ew, w1, b1, w2, b2):
    e, pos_w = posd.shape
    l_num, _, h_dim = w1.shape
    be = 4000 if e % 4000 == 0 else e

    def body(pd_ref, ps_ref, ew_ref, w1_ref, b1_ref, w2_ref, b2_ref, o_ref):
        t = (jnp.dot(pd_ref[...], w1_ref[0, :pos_w],
                     preferred_element_type=jnp.float32)
             + jnp.dot(ps_ref[...], w1_ref[0, pos_w:],
                       preferred_element_type=jnp.float32)
             + b1_ref[0])
        t = jax.nn.gelu(t)
        k = jnp.dot(t, w2_ref[0], preferred_element_type=jnp.float32) + b2_ref[0]
        o_ref[0] = k * ew_ref[...]

    return pl.pallas_call(
        body,
        grid=(l_num, e // be),
        in_specs=[
            pl.BlockSpec((be, pos_w), lambda l, i: (i, 0)),
            pl.BlockSpec((be, pos_w), lambda l, i: (i, 0)),
            pl.BlockSpec((be, 1), lambda l, i: (i, 0)),
            pl.BlockSpec((1, 2 * pos_w, h_dim), lambda l, i: (l, 0, 0)),
            pl.BlockSpec((1, 1, h_dim), lambda l, i: (l, 0, 0)),
            pl.BlockSpec((1, h_dim, h_dim), lambda l, i: (l, 0, 0)),
            pl.BlockSpec((1, 1, h_dim), lambda l, i: (l, 0, 0)),
        ],
        out_specs=pl.BlockSpec((1, be, h_dim), lambda l, i: (l, i, 0)),
        out_shape=jax.ShapeDtypeStruct((l_num, e, h_dim), jnp.float32),
    )(posd, poss, ew.reshape(e, 1), w1, b1.reshape(l_num, 1, h_dim), w2,
      b2.reshape(l_num, 1, h_dim))


def _tc_combine(aggp, degp, h, proj_w=None, proj_b=None):
    n, h_dim = h.shape
    bn = 2000 if n % 2000 == 0 else n
    final = proj_w is not None

    def body(*refs):
        if final:
            a_ref, d_ref, h_ref, pw_ref, pb_ref, o_ref = refs
        else:
            a_ref, d_ref, h_ref, o_ref = refs
        agg = a_ref[0] + a_ref[1]
        deg = jnp.clip(d_ref[0, :, 0:1] + d_ref[1, :, 0:1], 1.0, None)
        hn = jax.nn.gelu(agg / deg + h_ref[...])
        if final:
            o_ref[...] = (jnp.dot(hn, pw_ref[...],
                                  preferred_element_type=jnp.float32)
                          + pb_ref[...])
        else:
            o_ref[...] = hn

    in_specs = [
        pl.BlockSpec((NC, bn, h_dim), lambda i: (0, i, 0)),
        pl.BlockSpec((NC, bn, LANES), lambda i: (0, i, 0)),
        pl.BlockSpec((bn, h_dim), lambda i: (i, 0)),
    ]
    args = [aggp, degp, h]
    if final:
        out_dim = proj_w.shape[1]
        in_specs += [pl.BlockSpec(proj_w.shape, lambda i: (0, 0)),
                     pl.BlockSpec((1, out_dim), lambda i: (0, 0))]
        args += [proj_w, proj_b.reshape(1, out_dim)]
        out_shape = jax.ShapeDtypeStruct((n, out_dim), jnp.float32)
        out_spec = pl.BlockSpec((bn, out_dim), lambda i: (i, 0))
    else:
        out_shape = jax.ShapeDtypeStruct((n, h_dim), jnp.float32)
        out_spec = pl.BlockSpec((bn, h_dim), lambda i: (i, 0))

    return pl.pallas_call(
        body, grid=(n // bn,), in_specs=in_specs, out_specs=out_spec,
        out_shape=out_shape,
    )(*args)


def kernel(x, pos, edge_index, edge_weights, lift_W, lift_b,
           ker_W1, ker_b1, ker_W2, ker_b2, proj_W, proj_b):
    b, n, c_in = x.shape
    e = edge_index.shape[1]
    h_dim = lift_W.shape[1]
    l_num = ker_W1.shape[0]
    n_pad = ((n + NS * CH - 1) // (NS * CH)) * (NS * CH)

    src = edge_index[0].astype(jnp.int32)
    dst = edge_index[1].astype(jnp.int32)
    x2d = x.reshape(n, c_in)

    h0 = _tc_lift(x2d, pos, lift_W[:c_in], lift_W[c_in:], lift_b)
    posd, poss, degp = _sc_prep(pos, src, dst, n_pad)
    kw = _tc_kmlp(posd, poss, edge_weights, ker_W1, ker_b1, ker_W2, ker_b2)

    h = h0
    for l in range(l_num):
        aggp = _sc_layer(h, kw[l], src, dst, n_pad)
        if l < l_num - 1:
            h = _tc_combine(aggp, degp, h)
        else:
            out = _tc_combine(aggp, degp, h, proj_W, proj_b)
    return out.reshape(b, 1, n) if proj_W.shape[1] == 1 else (
        out.reshape(b, n, proj_W.shape[1]).transpose(0, 2, 1))
